# Initial kernel scaffold; baseline (speedup 1.0000x reference)
import jax
import jax.numpy as jnp
from jax.experimental import pallas as pl


def kernel(input_matrix, adj, W1, a1_src, a1_dst, W2, a2_src, a2_dst, W3, a3_src, a3_dst):
    raise NotImplementedError("write your pallas kernel here")


# speedup vs baseline: 1.0534x; 1.0534x over previous
import functools

import jax
import jax.numpy as jnp
from jax.experimental import pallas as pl
from jax.experimental.pallas import tpu as pltpu

N = 10000
E = 160000


def _mm_body(x_ref, w_ref, o_ref):
    o_ref[...] = jnp.dot(x_ref[...], w_ref[...],
                         preferred_element_type=jnp.float32)


def _matmul(x, w, bm=1000):
    m, k = x.shape
    k2, n = w.shape
    grid = (m // bm,)
    return pl.pallas_call(
        _mm_body,
        grid=grid,
        in_specs=[
            pl.BlockSpec((bm, k), lambda i: (i, 0)),
            pl.BlockSpec((k, n), lambda i: (0, 0)),
        ],
        out_specs=pl.BlockSpec((bm, n), lambda i: (i, 0)),
        out_shape=jax.ShapeDtypeStruct((m, n), jnp.float32),
    )(x, w)


def _gat_layer(x, src, dst, W, a_src, a_dst, heads, d_head, concat):
    n = x.shape[0]
    h = _matmul(x, W)
    h3 = h.reshape(n, heads, d_head)
    alpha_src = jnp.sum(h3 * a_src[None, :, :], axis=-1)
    alpha_dst = jnp.sum(h3 * a_dst[None, :, :], axis=-1)
    shift = jnp.max(alpha_src, axis=0) + jnp.max(alpha_dst, axis=0)
    e = jax.nn.leaky_relu(alpha_src[src] + alpha_dst[dst], negative_slope=0.2)
    p = jnp.exp(e - shift[None, :])
    denom = jax.ops.segment_sum(p, dst, num_segments=n)
    msg = h3[src] * p[:, :, None]
    out = jax.ops.segment_sum(msg, dst, num_segments=n)
    out = out / (denom[:, :, None] + 1e-16)
    if concat:
        return out.reshape(n, heads * d_head)
    return out.mean(axis=1)


def kernel(input_matrix, adj, W1, a1_src, a1_dst, W2, a2_src, a2_dst,
           W3, a3_src, a3_dst):
    src = adj[0]
    dst = adj[1]
    x = _gat_layer(input_matrix, src, dst, W1, a1_src, a1_dst, 4, 256, True)
    x = jax.nn.elu(x)
    y = _gat_layer(x, src, dst, W2, a2_src, a2_dst, 4, 256, True)
    x = jax.nn.elu(y) + x
    x = _gat_layer(x, src, dst, W3, a3_src, a3_dst, 6, 121, False)
    return x


# speedup vs baseline: 7.9586x; 7.5549x over previous
import functools

import jax
import jax.numpy as jnp
from jax import lax
from jax.experimental import pallas as pl
from jax.experimental.pallas import tpu as pltpu
from jax.experimental.pallas import tpu_sc as plsc

N = 10000
NP = 10240
E = 160000
NS = 16
NC = 2
EPW = E // NS
KC = 80
NCH = EPW // KC
NBLK = 5
CPB = NCH // NBLK
NH = 8
NHPS = NH // NC
RPW = NP // NS
_ZCNT = (80,) * 8

f32 = jnp.float32


def _tc_layer(x_or_raw, den, res, W, A, bm=512):
    n, din = x_or_raw.shape
    dout = W.shape[1]
    prologue = den is not None

    def body(*refs):
        if prologue:
            if res is not None:
                raw_ref, den_ref, res_ref, w_ref, a_ref = refs[:5]
                xo_ref, h_ref, as_ref, sh_ref = refs[5:]
            else:
                raw_ref, den_ref, w_ref, a_ref = refs[:4]
                xo_ref, h_ref, as_ref, sh_ref = refs[4:]
            d = den_ref[...]
            parts = []
            for hh in range(8):
                col = 128 * (hh // 4) + 16 * (hh % 4)
                parts.append(raw_ref[:, hh * 128:(hh + 1) * 128]
                             / (d[:, col:col + 1] + 1e-16))
            v = jnp.concatenate(parts, axis=1)
            x = jnp.where(v > 0.0, v, jnp.exp(v) - 1.0)
            if res is not None:
                x = x + res_ref[...]
            xo_ref[...] = x
        else:
            x_ref, w_ref, a_ref, h_ref, as_ref, sh_ref = refs
            x = x_ref[...]
        i = pl.program_id(0)
        h = jnp.dot(x, w_ref[...], preferred_element_type=f32)
        h_ref[...] = h
        asad = jnp.dot(h, a_ref[...], preferred_element_type=f32)
        as_ref[...] = asad
        m = jnp.max(asad, axis=0, keepdims=True)

        @pl.when(i == 0)
        def _():
            sh_ref[...] = m

        @pl.when(i != 0)
        def _():
            sh_ref[...] = jnp.maximum(sh_ref[...], m)

    grid = (n // bm,)
    in_specs = []
    ins = []
    if prologue:
        in_specs.append(pl.BlockSpec((bm, din), lambda i: (i, 0)))
        ins.append(x_or_raw)
        in_specs.append(pl.BlockSpec((bm, 256), lambda i: (i, 0)))
        ins.append(den)
        if res is not None:
            in_specs.append(pl.BlockSpec((bm, din), lambda i: (i, 0)))
            ins.append(res)
    else:
        in_specs.append(pl.BlockSpec((bm, din), lambda i: (i, 0)))
        ins.append(x_or_raw)
    in_specs.append(pl.BlockSpec((din, dout), lambda i: (0, 0)))
    ins.append(W)
    in_specs.append(pl.BlockSpec((dout, 16), lambda i: (0, 0)))
    ins.append(A)

    out_specs = []
    out_shape = []
    if prologue:
        out_specs.append(pl.BlockSpec((bm, din), lambda i: (i, 0)))
        out_shape.append(jax.ShapeDtypeStruct((n, din), f32))
    out_specs.append(pl.BlockSpec((bm, dout), lambda i: (i, 0)))
    out_shape.append(jax.ShapeDtypeStruct((n, dout), f32))
    out_specs.append(pl.BlockSpec((bm, 16), lambda i: (i, 0)))
    out_shape.append(jax.ShapeDtypeStruct((n, 16), f32))
    out_specs.append(pl.BlockSpec((1, 16), lambda i: (0, 0)))
    out_shape.append(jax.ShapeDtypeStruct((1, 16), f32))

    outs = pl.pallas_call(body, grid=grid, in_specs=in_specs,
                          out_specs=out_specs, out_shape=out_shape)(*ins)
    if prologue:
        x, h, asad, shift = outs
    else:
        h, asad, shift = outs
        x = x_or_raw
    return x, h, asad, shift


def _tc_final(raw, den, bm=512):
    def body(raw_ref, den_ref, o_ref):
        d = den_ref[...]
        acc = raw_ref[:, 0:128] / (d[:, 0:1] + 1e-16)
        for hh in range(1, 6):
            col = 128 * (hh // 4) + 16 * (hh % 4)
            acc = acc + (raw_ref[:, hh * 128:(hh + 1) * 128]
                         / (d[:, col:col + 1] + 1e-16))
        o_ref[...] = acc * (1.0 / 6.0)

    return pl.pallas_call(
        body, grid=(NP // bm,),
        in_specs=[pl.BlockSpec((bm, 1024), lambda i: (i, 0)),
                  pl.BlockSpec((bm, 256), lambda i: (i, 0))],
        out_specs=pl.BlockSpec((bm, 128), lambda i: (i, 0)),
        out_shape=jax.ShapeDtypeStruct((NP, 128), f32),
    )(raw, den)


HN = 5120
TRASH = HN
HRW = HN // NS


def _make_sc():
    mesh = plsc.VectorSubcoreMesh(core_axis_name="c", subcore_axis_name="s")
    G = 5
    CG = CPB // G

    @functools.partial(
        pl.kernel,
        out_type=[jax.ShapeDtypeStruct((NH, NP, 128), f32),
                  jax.ShapeDtypeStruct((NC, NP, 128), f32),
                  jax.ShapeDtypeStruct((NH, NS, NBLK, G, CG, KC), f32)],
        mesh=mesh,
        compiler_params=pltpu.CompilerParams(needs_layout_passes=False),
        scratch_types=[
            pltpu.VMEM((CPB, KC), jnp.int32),
            pltpu.VMEM((CPB, KC), jnp.int32),
            pltpu.VMEM((1, KC), jnp.int32),
            pltpu.VMEM((CG, KC), f32),
            pltpu.VMEM((CG, KC), f32),
            pltpu.VMEM((CG, KC), f32),
            pltpu.VMEM((CG, KC), f32),
            pltpu.VMEM((CG, KC), f32),
            pltpu.VMEM((NP,), f32),
            pltpu.VMEM((NP,), f32),
            pltpu.VMEM((KC,), jnp.int32),
            pltpu.VMEM((KC, 128), f32),
            pltpu.VMEM((16,), f32),
            pltpu.VMEM_SHARED((HN + 8, 128), f32),
        ],
    )
    def k(h_flat, asadT, shift_in, src4, dst4, out_raw, den_out, p_out,
          src25, dst25, sdx, pexp, pd0, pd1, pd2, pd3,
          as_buf, ad_buf, idx_buf, rows, shiftv, slab):
        c = lax.axis_index("c")
        s = lax.axis_index("s")
        pltpu.sync_copy(shift_in, shiftv)
        pib = "promise_in_bounds"
        pd = [pd0, pd1, pd2, pd3]

        def zero_rows(r, t):
            for j in range(8):
                rows[r, pl.ds(16 * j, 16)] = jnp.zeros((16,), f32)
            return t

        def zero_own_slab():
            lax.fori_loop(0, KC, zero_rows, 0)
            off = 0
            for _ in range(HRW // KC):
                pltpu.sync_copy(rows.at[pl.ds(0, KC)],
                                slab.at[pl.ds(HRW * s + off, KC)])
                off += KC

            @pl.when(s == NS - 1)
            def _():
                pltpu.sync_copy(rows.at[pl.ds(0, 8)],
                                slab.at[pl.ds(HN, 8)])

        def leaky_exp(sv, dv, shc):
            a1 = plsc.load_gather(as_buf, [sv])
            a2 = plsc.load_gather(ad_buf, [dv])
            e = a1 + a2
            e = jnp.where(e >= 0.0, e, 0.2 * e)
            return jnp.exp(e - shc)

        for fci in range(NHPS):
            fc = c * NHPS + fci
            fcv = jnp.full((16,), fc, jnp.int32)
            shv = shiftv[pl.ds(0, 16)]
            shc = jnp.maximum(shv.at[fcv].get(mode=pib)
                              + shv.at[fcv + 8].get(mode=pib), 0.0)
            pltpu.sync_copy(asadT.at[pl.ds(fc * NP, NP)], as_buf)
            pltpu.sync_copy(asadT.at[pl.ds((8 + fc) * NP, NP)], ad_buf)
            for half in range(NP // HN):
                base = half * HN
                zero_own_slab()
                plsc.subcore_barrier()

                def blk_loop(blk, t0, half=half, fc=fc, shc=shc, base=base):
                    pltpu.sync_copy(src4.at[s, blk], src25)
                    pltpu.sync_copy(dst4.at[s, blk], dst25)

                    def g_loop(g, t1):
                        if half == 1:
                            pltpu.sync_copy(p_out.at[fc, s, blk, g], pexp)

                        def p2_body(rr, t2):
                            cc = CG * g + rr
                            for j in range(KC // 16):
                                sv = src25[cc, pl.ds(16 * j, 16)]
                                dv = dst25[cc, pl.ds(16 * j, 16)]
                                idx_buf[pl.ds(16 * j, 16)] = sv * NH + fc
                                dl = dv - base
                                ok = (dl >= 0) & (dl < HN)
                                sdx[0, pl.ds(16 * j, 16)] = jnp.where(
                                    ok, dl, TRASH)
                                if half == 0:
                                    pexp[rr, pl.ds(16 * j, 16)] = leaky_exp(
                                        sv, dv, shc)
                            pltpu.sync_copy(h_flat.at[idx_buf], rows)

                            def sk(jj, tt):
                                pv = pexp[rr, pl.ds(16 * jj, 16)]
                                for k in range(16):
                                    ps = pv[k]
                                    kk = 16 * jj + k
                                    for j in range(8):
                                        rows[kk, pl.ds(16 * j, 16)] = (
                                            rows[kk, pl.ds(16 * j, 16)]
                                            * ps)
                                return tt

                            lax.fori_loop(0, KC // 16, sk, 0)
                            pltpu.sync_copy(rows, slab.at[sdx.at[0]],
                                            add=True)
                            return t2

                        lax.fori_loop(0, CG, p2_body, 0)
                        if half == 0:
                            pltpu.sync_copy(pexp, p_out.at[fc, s, blk, g])
                        return t1

                    lax.fori_loop(0, G, g_loop, 0)
                    return t0

                lax.fori_loop(0, NBLK, blk_loop, 0)

                plsc.subcore_barrier()
                pltpu.sync_copy(
                    slab.at[pl.ds(HRW * s, HRW)],
                    out_raw.at[fc, pl.ds(base + HRW * s, HRW)])
                plsc.subcore_barrier()

        for half in range(NP // HN):
            base = half * HN
            zero_own_slab()
            plsc.subcore_barrier()

            def dblk_loop(blk, t0, base=base):
                pltpu.sync_copy(dst4.at[s, blk], dst25)

                def dg_loop(g, t1):
                    for hl in range(NHPS):
                        pltpu.sync_copy(p_out.at[c * NHPS + hl, s, blk, g],
                                        pd[hl])

                    def d_body(rr, t2):
                        cc = CG * g + rr
                        for jj in range(KC // 16):
                            pvs = [pd[hl][rr, pl.ds(16 * jj, 16)]
                                   for hl in range(NHPS)]
                            for k in range(16):
                                kk = 16 * jj + k
                                for hl in range(NHPS):
                                    rows[kk, pl.ds(16 * hl, 16)] = jnp.full(
                                        (16,), pvs[hl][k], f32)
                            dv = dst25[cc, pl.ds(16 * jj, 16)]
                            dl = dv - base
                            ok = (dl >= 0) & (dl < HN)
                            sdx[0, pl.ds(16 * jj, 16)] = jnp.where(
                                ok, dl, TRASH)
                        pltpu.sync_copy(rows, slab.at[sdx.at[0]], add=True)
                        return t2

                    lax.fori_loop(0, CG, d_body, 0)
                    return t1

                lax.fori_loop(0, G, dg_loop, 0)
                return t0

            lax.fori_loop(0, NBLK, dblk_loop, 0)

            plsc.subcore_barrier()
            pltpu.sync_copy(
                slab.at[pl.ds(HRW * s, HRW)],
                den_out.at[c, pl.ds(base + HRW * s, HRW)])
            plsc.subcore_barrier()

    return k


@functools.lru_cache(maxsize=None)
def _sc_kernel():
    return _make_sc()


def _attn_pseudo(cols_src, cols_dst, dout):
    A = jnp.zeros((dout, 16), f32)
    for f in range(8):
        if cols_src[f] is not None:
            r0, v = cols_src[f]
            A = A.at[r0:r0 + v.shape[0], f].set(v)
        if cols_dst[f] is not None:
            r0, v = cols_dst[f]
            A = A.at[r0:r0 + v.shape[0], 8 + f].set(v)
    return A


def _attn_mat4(a_src, a_dst):
    cs = [(256 * (f // 2), a_src[f // 2]) for f in range(8)]
    cd = [(256 * (f // 2), a_dst[f // 2]) for f in range(8)]
    return _attn_pseudo(cs, cd, 1024)


def _attn_mat6(a_src, a_dst):
    cs = [(128 * f, a_src[f]) if f < 6 else None for f in range(8)]
    cd = [(128 * f, a_dst[f]) if f < 6 else None for f in range(8)]
    return _attn_pseudo(cs, cd, 1024)


def kernel(input_matrix, adj, W1, a1_src, a1_dst, W2, a2_src, a2_dst,
           W3, a3_src, a3_dst):
    src4 = adj[0].astype(jnp.int32).reshape(NS, NBLK, CPB, KC)
    dst4 = adj[1].astype(jnp.int32).reshape(NS, NBLK, CPB, KC)

    A1 = _attn_mat4(a1_src, a1_dst)
    A2 = _attn_mat4(a2_src, a2_dst)
    A3 = _attn_mat6(a3_src, a3_dst)
    W3p = jnp.pad(W3.reshape(1024, 6, 121),
                  ((0, 0), (0, 0), (0, 7)))
    W3p = jnp.pad(W3p.reshape(1024, 768), ((0, 0), (0, 256)))
    xp = jnp.pad(input_matrix, ((0, NP - N), (0, 0)))

    sc = _sc_kernel()

    def tview(a):
        return a.T.reshape(16 * NP)

    def nview(raw):
        return jnp.transpose(raw, (1, 0, 2)).reshape(NP, 1024)

    def dview(dcn):
        return jnp.concatenate([dcn[0], dcn[1]], axis=1)

    _, h1, asad1, sh1 = _tc_layer(xp, None, None, W1, A1)
    raw1, den1, _p = sc(h1.reshape(NP * 8, 128), tview(asad1),
                        sh1.reshape(16), src4, dst4)
    x1, h2, asad2, sh2 = _tc_layer(nview(raw1), dview(den1), None, W2, A2)
    raw2, den2, _p = sc(h2.reshape(NP * 8, 128), tview(asad2),
                        sh2.reshape(16), src4, dst4)
    _, h3, asad3, sh3 = _tc_layer(nview(raw2), dview(den2), x1, W3p, A3)
    raw3, den3, _p = sc(h3.reshape(NP * 8, 128), tview(asad3),
                        sh3.reshape(16), src4, dst4)
    out = _tc_final(nview(raw3), dview(den3))
    return out[:N, :121]


# speedup vs baseline: 10.6626x; 1.3398x over previous
import functools

import jax
import jax.numpy as jnp
from jax import lax
from jax.experimental import pallas as pl
from jax.experimental.pallas import tpu as pltpu
from jax.experimental.pallas import tpu_sc as plsc

N = 10000
NP = 10240
E = 160000
NS = 16
NC = 2
EPW = E // NS
KC = 80
NCH = EPW // KC
NBLK = 5
CPB = NCH // NBLK
NH = 8
NHPS = NH // NC
RPW = NP // NS
_ZCNT = (80,) * 8

f32 = jnp.float32


def _tc_layer(x_or_raw, den, res, W, A, bm=512):
    n, din = x_or_raw.shape
    dout = W.shape[1]
    prologue = den is not None

    def body(*refs):
        if prologue:
            if res is not None:
                raw_ref, den_ref, res_ref, w_ref, a_ref = refs[:5]
                xo_ref, h_ref, as_ref, sh_ref = refs[5:]
            else:
                raw_ref, den_ref, w_ref, a_ref = refs[:4]
                xo_ref, h_ref, as_ref, sh_ref = refs[4:]
            d = den_ref[...]
            parts = []
            for hh in range(8):
                col = 128 * (hh // 4) + 16 * (hh % 4)
                parts.append(raw_ref[:, hh * 128:(hh + 1) * 128]
                             / (d[:, col:col + 1] + 1e-16))
            v = jnp.concatenate(parts, axis=1)
            x = jnp.where(v > 0.0, v, jnp.exp(v) - 1.0)
            if res is not None:
                x = x + res_ref[...]
            xo_ref[...] = x
        else:
            x_ref, w_ref, a_ref, h_ref, as_ref, sh_ref = refs
            x = x_ref[...]
        i = pl.program_id(0)
        h = jnp.dot(x, w_ref[...], preferred_element_type=f32)
        h_ref[...] = h
        asad = jnp.dot(h, a_ref[...], preferred_element_type=f32)
        as_ref[...] = asad
        m = jnp.max(asad, axis=0, keepdims=True)

        @pl.when(i == 0)
        def _():
            sh_ref[...] = m

        @pl.when(i != 0)
        def _():
            sh_ref[...] = jnp.maximum(sh_ref[...], m)

    grid = (n // bm,)
    in_specs = []
    ins = []
    if prologue:
        in_specs.append(pl.BlockSpec((bm, din), lambda i: (i, 0)))
        ins.append(x_or_raw)
        in_specs.append(pl.BlockSpec((bm, 256), lambda i: (i, 0)))
        ins.append(den)
        if res is not None:
            in_specs.append(pl.BlockSpec((bm, din), lambda i: (i, 0)))
            ins.append(res)
    else:
        in_specs.append(pl.BlockSpec((bm, din), lambda i: (i, 0)))
        ins.append(x_or_raw)
    in_specs.append(pl.BlockSpec((din, dout), lambda i: (0, 0)))
    ins.append(W)
    in_specs.append(pl.BlockSpec((dout, 16), lambda i: (0, 0)))
    ins.append(A)

    out_specs = []
    out_shape = []
    if prologue:
        out_specs.append(pl.BlockSpec((bm, din), lambda i: (i, 0)))
        out_shape.append(jax.ShapeDtypeStruct((n, din), f32))
    out_specs.append(pl.BlockSpec((bm, dout), lambda i: (i, 0)))
    out_shape.append(jax.ShapeDtypeStruct((n, dout), f32))
    out_specs.append(pl.BlockSpec((bm, 16), lambda i: (i, 0)))
    out_shape.append(jax.ShapeDtypeStruct((n, 16), f32))
    out_specs.append(pl.BlockSpec((1, 16), lambda i: (0, 0)))
    out_shape.append(jax.ShapeDtypeStruct((1, 16), f32))

    outs = pl.pallas_call(body, grid=grid, in_specs=in_specs,
                          out_specs=out_specs, out_shape=out_shape)(*ins)
    if prologue:
        x, h, asad, shift = outs
    else:
        h, asad, shift = outs
        x = x_or_raw
    return x, h, asad, shift


def _tc_final(raw, den, bm=512):
    def body(raw_ref, den_ref, o_ref):
        d = den_ref[...]
        acc = raw_ref[:, 0:128] / (d[:, 0:1] + 1e-16)
        for hh in range(1, 6):
            col = 128 * (hh // 4) + 16 * (hh % 4)
            acc = acc + (raw_ref[:, hh * 128:(hh + 1) * 128]
                         / (d[:, col:col + 1] + 1e-16))
        o_ref[...] = acc * (1.0 / 6.0)

    return pl.pallas_call(
        body, grid=(NP // bm,),
        in_specs=[pl.BlockSpec((bm, 1024), lambda i: (i, 0)),
                  pl.BlockSpec((bm, 256), lambda i: (i, 0))],
        out_specs=pl.BlockSpec((bm, 128), lambda i: (i, 0)),
        out_shape=jax.ShapeDtypeStruct((NP, 128), f32),
    )(raw, den)


HN = 5120
TRASH = HN
HRW = HN // NS


def _make_sc():
    mesh = plsc.VectorSubcoreMesh(core_axis_name="c", subcore_axis_name="s")
    G = 5
    CG = CPB // G

    @functools.partial(
        pl.kernel,
        out_type=[jax.ShapeDtypeStruct((NH, NP, 128), f32),
                  jax.ShapeDtypeStruct((NC, NP, 128), f32),
                  jax.ShapeDtypeStruct((NH, NS, NBLK, G, CG, KC), f32)],
        mesh=mesh,
        compiler_params=pltpu.CompilerParams(needs_layout_passes=False),
        scratch_types=[
            pltpu.VMEM((CPB, KC), jnp.int32),
            pltpu.VMEM((CPB, KC), jnp.int32),
            pltpu.VMEM((1, KC), jnp.int32),
            pltpu.VMEM((CG, KC), f32),
            pltpu.VMEM((CG, KC), f32),
            pltpu.VMEM((CG, KC), f32),
            pltpu.VMEM((CG, KC), f32),
            pltpu.VMEM((CG, KC), f32),
            pltpu.VMEM((NP,), f32),
            pltpu.VMEM((NP,), f32),
            pltpu.VMEM((2, KC), jnp.int32),
            pltpu.VMEM((2 * KC, 128), f32),
            pltpu.VMEM((16,), f32),
            pltpu.SemaphoreType.DMA((2,)),
            pltpu.VMEM_SHARED((HN + 8, 128), f32),
        ],
    )
    def k(h_flat, asadT, shift_in, src4, dst4, out_raw, den_out, p_out,
          src25, dst25, sdx, pexp, pd0, pd1, pd2, pd3,
          as_buf, ad_buf, idx3, rows3, shiftv, gsem, slab):
        c = lax.axis_index("c")
        s = lax.axis_index("s")
        pltpu.sync_copy(shift_in, shiftv)
        pib = "promise_in_bounds"
        pd = [pd0, pd1, pd2, pd3]

        def zero_rows(r, t):
            for j in range(8):
                rows3[r, pl.ds(16 * j, 16)] = jnp.zeros((16,), f32)
            return t

        def zero_own_slab():
            lax.fori_loop(0, KC, zero_rows, 0)
            off = 0
            for _ in range(HRW // KC):
                pltpu.sync_copy(rows3.at[pl.ds(0, KC)],
                                slab.at[pl.ds(HRW * s + off, KC)])
                off += KC

            @pl.when(s == NS - 1)
            def _():
                pltpu.sync_copy(rows3.at[pl.ds(0, 8)],
                                slab.at[pl.ds(HN, 8)])

        def leaky_exp(sv, dv, shc):
            a1 = plsc.load_gather(as_buf, [sv])
            a2 = plsc.load_gather(ad_buf, [dv])
            e = a1 + a2
            e = jnp.where(e >= 0.0, e, 0.2 * e)
            return jnp.exp(e - shc)

        for fci in range(NHPS):
            fc = c * NHPS + fci
            fcv = jnp.full((16,), fc, jnp.int32)
            shv = shiftv[pl.ds(0, 16)]
            shc = jnp.maximum(shv.at[fcv].get(mode=pib)
                              + shv.at[fcv + 8].get(mode=pib), 0.0)
            pltpu.sync_copy(asadT.at[pl.ds(fc * NP, NP)], as_buf)
            pltpu.sync_copy(asadT.at[pl.ds((8 + fc) * NP, NP)], ad_buf)
            for half in range(NP // HN):
                base = half * HN
                zero_own_slab()
                plsc.subcore_barrier()

                def blk_loop(blk, t0, half=half, fc=fc, shc=shc, base=base):
                    pltpu.sync_copy(src4.at[s, blk], src25)
                    pltpu.sync_copy(dst4.at[s, blk], dst25)

                    def build_idx(cc, buf_row):
                        for j in range(KC // 16):
                            sv = src25[cc, pl.ds(16 * j, 16)]
                            idx3[buf_row, pl.ds(16 * j, 16)] = sv * NH + fc

                    def issue(buf_row):
                        pltpu.async_copy(
                            h_flat.at[idx3.at[buf_row]],
                            rows3.at[pl.ds(buf_row * KC, KC)],
                            gsem.at[buf_row])

                    def drain(buf_row):
                        pltpu.make_async_copy(
                            h_flat.at[idx3.at[buf_row]],
                            rows3.at[pl.ds(buf_row * KC, KC)],
                            gsem.at[buf_row]).wait()

                    build_idx(0, 0)
                    issue(0)

                    def p2_body(cc, t2):
                        par = lax.rem(cc, 2)
                        nxt = 1 - par
                        g = lax.div(cc, CG)
                        rr = lax.rem(cc, CG)

                        @pl.when(cc < CPB - 1)
                        def _():
                            build_idx(cc + 1, nxt)
                            issue(nxt)

                        if half == 1:
                            @pl.when(rr == 0)
                            def _():
                                pltpu.sync_copy(
                                    p_out.at[fc, s, blk, g], pexp)
                        for j in range(KC // 16):
                            dv = dst25[cc, pl.ds(16 * j, 16)]
                            dl = dv - base
                            ok = (dl >= 0) & (dl < HN)
                            sdx[0, pl.ds(16 * j, 16)] = jnp.where(
                                ok, dl, TRASH)
                            if half == 0:
                                sv = src25[cc, pl.ds(16 * j, 16)]
                                pexp[rr, pl.ds(16 * j, 16)] = leaky_exp(
                                    sv, dv, shc)
                        drain(par)

                        def sk(jj, tt):
                            pv = pexp[rr, pl.ds(16 * jj, 16)]
                            for k in range(16):
                                ps = pv[k]
                                kk = 16 * jj + k
                                for j in range(8):
                                    rows3[par * KC + kk,
                                          pl.ds(16 * j, 16)] = (
                                        rows3[par * KC + kk,
                                              pl.ds(16 * j, 16)] * ps)
                            return tt

                        lax.fori_loop(0, KC // 16, sk, 0)
                        pltpu.sync_copy(rows3.at[pl.ds(par * KC, KC)],
                                        slab.at[sdx.at[0]], add=True)
                        if half == 0:
                            @pl.when(rr == CG - 1)
                            def _():
                                pltpu.sync_copy(
                                    pexp, p_out.at[fc, s, blk, g])
                        return t2

                    lax.fori_loop(0, CPB, p2_body, 0)
                    return t0

                lax.fori_loop(0, NBLK, blk_loop, 0)

                plsc.subcore_barrier()
                pltpu.sync_copy(
                    slab.at[pl.ds(HRW * s, HRW)],
                    out_raw.at[fc, pl.ds(base + HRW * s, HRW)])
                plsc.subcore_barrier()

        for half in range(NP // HN):
            base = half * HN
            zero_own_slab()
            plsc.subcore_barrier()

            def dblk_loop(blk, t0, base=base):
                pltpu.sync_copy(dst4.at[s, blk], dst25)

                def dg_loop(g, t1):
                    for hl in range(NHPS):
                        pltpu.sync_copy(p_out.at[c * NHPS + hl, s, blk, g],
                                        pd[hl])

                    def d_body(rr, t2):
                        cc = CG * g + rr
                        for jj in range(KC // 16):
                            pvs = [pd[hl][rr, pl.ds(16 * jj, 16)]
                                   for hl in range(NHPS)]
                            for k in range(16):
                                kk = 16 * jj + k
                                for hl in range(NHPS):
                                    rows3[kk, pl.ds(16 * hl, 16)] = (
                                        jnp.full((16,), pvs[hl][k], f32))
                            dv = dst25[cc, pl.ds(16 * jj, 16)]
                            dl = dv - base
                            ok = (dl >= 0) & (dl < HN)
                            sdx[0, pl.ds(16 * jj, 16)] = jnp.where(
                                ok, dl, TRASH)
                        pltpu.sync_copy(rows3.at[pl.ds(0, KC)],
                                        slab.at[sdx.at[0]], add=True)
                        return t2

                    lax.fori_loop(0, CG, d_body, 0)
                    return t1

                lax.fori_loop(0, G, dg_loop, 0)
                return t0

            lax.fori_loop(0, NBLK, dblk_loop, 0)

            plsc.subcore_barrier()
            pltpu.sync_copy(
                slab.at[pl.ds(HRW * s, HRW)],
                den_out.at[c, pl.ds(base + HRW * s, HRW)])
            plsc.subcore_barrier()

    return k


@functools.lru_cache(maxsize=None)
def _sc_kernel():
    return _make_sc()


def _attn_pseudo(cols_src, cols_dst, dout):
    A = jnp.zeros((dout, 16), f32)
    for f in range(8):
        if cols_src[f] is not None:
            r0, v = cols_src[f]
            A = A.at[r0:r0 + v.shape[0], f].set(v)
        if cols_dst[f] is not None:
            r0, v = cols_dst[f]
            A = A.at[r0:r0 + v.shape[0], 8 + f].set(v)
    return A


def _attn_mat4(a_src, a_dst):
    cs = [(256 * (f // 2), a_src[f // 2]) for f in range(8)]
    cd = [(256 * (f // 2), a_dst[f // 2]) for f in range(8)]
    return _attn_pseudo(cs, cd, 1024)


def _attn_mat6(a_src, a_dst):
    cs = [(128 * f, a_src[f]) if f < 6 else None for f in range(8)]
    cd = [(128 * f, a_dst[f]) if f < 6 else None for f in range(8)]
    return _attn_pseudo(cs, cd, 1024)


def kernel(input_matrix, adj, W1, a1_src, a1_dst, W2, a2_src, a2_dst,
           W3, a3_src, a3_dst):
    src4 = adj[0].astype(jnp.int32).reshape(NS, NBLK, CPB, KC)
    dst4 = adj[1].astype(jnp.int32).reshape(NS, NBLK, CPB, KC)

    A1 = _attn_mat4(a1_src, a1_dst)
    A2 = _attn_mat4(a2_src, a2_dst)
    A3 = _attn_mat6(a3_src, a3_dst)
    W3p = jnp.pad(W3.reshape(1024, 6, 121),
                  ((0, 0), (0, 0), (0, 7)))
    W3p = jnp.pad(W3p.reshape(1024, 768), ((0, 0), (0, 256)))
    xp = jnp.pad(input_matrix, ((0, NP - N), (0, 0)))

    sc = _sc_kernel()

    def tview(a):
        return a.T.reshape(16 * NP)

    def nview(raw):
        return jnp.transpose(raw, (1, 0, 2)).reshape(NP, 1024)

    def dview(dcn):
        return jnp.concatenate([dcn[0], dcn[1]], axis=1)

    _, h1, asad1, sh1 = _tc_layer(xp, None, None, W1, A1)
    raw1, den1, _p = sc(h1.reshape(NP * 8, 128), tview(asad1),
                        sh1.reshape(16), src4, dst4)
    x1, h2, asad2, sh2 = _tc_layer(nview(raw1), dview(den1), None, W2, A2)
    raw2, den2, _p = sc(h2.reshape(NP * 8, 128), tview(asad2),
                        sh2.reshape(16), src4, dst4)
    _, h3, asad3, sh3 = _tc_layer(nview(raw2), dview(den2), x1, W3p, A3)
    raw3, den3, _p = sc(h3.reshape(NP * 8, 128), tview(asad3),
                        sh3.reshape(16), src4, dst4)
    out = _tc_final(nview(raw3), dview(den3))
    return out[:N, :121]


# speedup vs baseline: 10.8070x; 1.0135x over previous
import functools

import jax
import jax.numpy as jnp
from jax import lax
from jax.experimental import pallas as pl
from jax.experimental.pallas import tpu as pltpu
from jax.experimental.pallas import tpu_sc as plsc

N = 10000
NP = 10240
E = 160000
NS = 16
NC = 2
EPW = E // NS
KC = 80
NCH = EPW // KC
NBLK = 5
CPB = NCH // NBLK
NH = 8
NHPS = NH // NC
RPW = NP // NS
_ZCNT = (80,) * 8

f32 = jnp.float32


def _tc_layer(x_or_raw, den, res, W, A, bm=512):
    n, din = x_or_raw.shape
    dout = W.shape[1]
    prologue = den is not None

    def body(*refs):
        if prologue:
            if res is not None:
                raw_ref, den_ref, res_ref, w_ref, a_ref = refs[:5]
                xo_ref, h_ref, as_ref, sh_ref = refs[5:]
            else:
                raw_ref, den_ref, w_ref, a_ref = refs[:4]
                xo_ref, h_ref, as_ref, sh_ref = refs[4:]
            d = den_ref[...]
            parts = []
            for hh in range(8):
                col = 128 * (hh // 4) + 16 * (hh % 4)
                parts.append(raw_ref[:, hh * 128:(hh + 1) * 128]
                             / (d[:, col:col + 1] + 1e-16))
            v = jnp.concatenate(parts, axis=1)
            x = jnp.where(v > 0.0, v, jnp.exp(v) - 1.0)
            if res is not None:
                x = x + res_ref[...]
            xo_ref[...] = x
        else:
            x_ref, w_ref, a_ref, h_ref, as_ref, sh_ref = refs
            x = x_ref[...]
        i = pl.program_id(0)
        h = jnp.dot(x, w_ref[...], preferred_element_type=f32)
        h_ref[...] = h
        asad = jnp.dot(h, a_ref[...], preferred_element_type=f32)
        as_ref[...] = asad
        m = jnp.max(asad, axis=0, keepdims=True)

        @pl.when(i == 0)
        def _():
            sh_ref[...] = m

        @pl.when(i != 0)
        def _():
            sh_ref[...] = jnp.maximum(sh_ref[...], m)

    grid = (n // bm,)
    in_specs = []
    ins = []
    if prologue:
        in_specs.append(pl.BlockSpec((bm, din), lambda i: (i, 0)))
        ins.append(x_or_raw)
        in_specs.append(pl.BlockSpec((bm, 256), lambda i: (i, 0)))
        ins.append(den)
        if res is not None:
            in_specs.append(pl.BlockSpec((bm, din), lambda i: (i, 0)))
            ins.append(res)
    else:
        in_specs.append(pl.BlockSpec((bm, din), lambda i: (i, 0)))
        ins.append(x_or_raw)
    in_specs.append(pl.BlockSpec((din, dout), lambda i: (0, 0)))
    ins.append(W)
    in_specs.append(pl.BlockSpec((dout, 16), lambda i: (0, 0)))
    ins.append(A)

    out_specs = []
    out_shape = []
    if prologue:
        out_specs.append(pl.BlockSpec((bm, din), lambda i: (i, 0)))
        out_shape.append(jax.ShapeDtypeStruct((n, din), f32))
    out_specs.append(pl.BlockSpec((bm, dout), lambda i: (i, 0)))
    out_shape.append(jax.ShapeDtypeStruct((n, dout), f32))
    out_specs.append(pl.BlockSpec((bm, 16), lambda i: (i, 0)))
    out_shape.append(jax.ShapeDtypeStruct((n, 16), f32))
    out_specs.append(pl.BlockSpec((1, 16), lambda i: (0, 0)))
    out_shape.append(jax.ShapeDtypeStruct((1, 16), f32))

    outs = pl.pallas_call(body, grid=grid, in_specs=in_specs,
                          out_specs=out_specs, out_shape=out_shape)(*ins)
    if prologue:
        x, h, asad, shift = outs
    else:
        h, asad, shift = outs
        x = x_or_raw
    return x, h, asad, shift


def _tc_final(raw, den, bm=512):
    def body(raw_ref, den_ref, o_ref):
        d = den_ref[...]
        acc = raw_ref[:, 0:128] / (d[:, 0:1] + 1e-16)
        for hh in range(1, 6):
            col = 128 * (hh // 4) + 16 * (hh % 4)
            acc = acc + (raw_ref[:, hh * 128:(hh + 1) * 128]
                         / (d[:, col:col + 1] + 1e-16))
        o_ref[...] = acc * (1.0 / 6.0)

    return pl.pallas_call(
        body, grid=(NP // bm,),
        in_specs=[pl.BlockSpec((bm, 1024), lambda i: (i, 0)),
                  pl.BlockSpec((bm, 256), lambda i: (i, 0))],
        out_specs=pl.BlockSpec((bm, 128), lambda i: (i, 0)),
        out_shape=jax.ShapeDtypeStruct((NP, 128), f32),
    )(raw, den)


HN = 5120
TRASH = HN
HRW = HN // NS


def _make_sc():
    mesh = plsc.VectorSubcoreMesh(core_axis_name="c", subcore_axis_name="s")
    G = 5
    CG = CPB // G

    @functools.partial(
        pl.kernel,
        out_type=[jax.ShapeDtypeStruct((NH, NP, 128), f32),
                  jax.ShapeDtypeStruct((NC, NP, 128), f32),
                  jax.ShapeDtypeStruct((NH, NS, NBLK, G, CG, KC), f32)],
        mesh=mesh,
        compiler_params=pltpu.CompilerParams(needs_layout_passes=False),
        scratch_types=[
            pltpu.VMEM((CPB, KC), jnp.int32),
            pltpu.VMEM((CPB, KC), jnp.int32),
            pltpu.VMEM((2, KC), jnp.int32),
            pltpu.VMEM((CG, KC), f32),
            pltpu.VMEM((CG, KC), f32),
            pltpu.VMEM((CG, KC), f32),
            pltpu.VMEM((CG, KC), f32),
            pltpu.VMEM((CG, KC), f32),
            pltpu.VMEM((NP,), f32),
            pltpu.VMEM((NP,), f32),
            pltpu.VMEM((2, KC), jnp.int32),
            pltpu.VMEM((2 * KC, 128), f32),
            pltpu.VMEM((16,), f32),
            pltpu.SemaphoreType.DMA((2,)),
            pltpu.SemaphoreType.DMA((2,)),
            pltpu.VMEM_SHARED((HN + 8, 128), f32),
        ],
    )
    def k(h_flat, asadT, shift_in, src4, dst4, out_raw, den_out, p_out,
          src25, dst25, sdx, pexp, pd0, pd1, pd2, pd3,
          as_buf, ad_buf, idx3, rows3, shiftv, gsem, ssem, slab):
        c = lax.axis_index("c")
        s = lax.axis_index("s")
        pltpu.sync_copy(shift_in, shiftv)
        pib = "promise_in_bounds"
        pd = [pd0, pd1, pd2, pd3]

        def zero_rows(r, t):
            for j in range(8):
                rows3[r, pl.ds(16 * j, 16)] = jnp.zeros((16,), f32)
            return t

        def zero_own_slab():
            lax.fori_loop(0, KC, zero_rows, 0)
            off = 0
            for _ in range(HRW // KC):
                pltpu.sync_copy(rows3.at[pl.ds(0, KC)],
                                slab.at[pl.ds(HRW * s + off, KC)])
                off += KC

            @pl.when(s == NS - 1)
            def _():
                pltpu.sync_copy(rows3.at[pl.ds(0, 8)],
                                slab.at[pl.ds(HN, 8)])

        def leaky_exp(sv, dv, shc):
            a1 = plsc.load_gather(as_buf, [sv])
            a2 = plsc.load_gather(ad_buf, [dv])
            e = a1 + a2
            e = jnp.where(e >= 0.0, e, 0.2 * e)
            return jnp.exp(e - shc)

        for fci in range(NHPS):
            fc = c * NHPS + fci
            fcv = jnp.full((16,), fc, jnp.int32)
            shv = shiftv[pl.ds(0, 16)]
            shc = jnp.maximum(shv.at[fcv].get(mode=pib)
                              + shv.at[fcv + 8].get(mode=pib), 0.0)
            pltpu.sync_copy(asadT.at[pl.ds(fc * NP, NP)], as_buf)
            pltpu.sync_copy(asadT.at[pl.ds((8 + fc) * NP, NP)], ad_buf)
            for half in range(NP // HN):
                base = half * HN
                zero_own_slab()
                plsc.subcore_barrier()

                def blk_loop(blk, t0, half=half, fc=fc, shc=shc, base=base):
                    pltpu.sync_copy(src4.at[s, blk], src25)
                    pltpu.sync_copy(dst4.at[s, blk], dst25)

                    def build_idx(cc, buf_row):
                        for j in range(KC // 16):
                            sv = src25[cc, pl.ds(16 * j, 16)]
                            idx3[buf_row, pl.ds(16 * j, 16)] = sv * NH + fc

                    def issue(buf_row):
                        pltpu.async_copy(
                            h_flat.at[idx3.at[buf_row]],
                            rows3.at[pl.ds(buf_row * KC, KC)],
                            gsem.at[buf_row])

                    def drain(buf_row):
                        pltpu.make_async_copy(
                            h_flat.at[idx3.at[buf_row]],
                            rows3.at[pl.ds(buf_row * KC, KC)],
                            gsem.at[buf_row]).wait()

                    def drain_scatter(buf_row):
                        pltpu.make_async_copy(
                            rows3.at[pl.ds(buf_row * KC, KC)],
                            slab.at[sdx.at[buf_row]],
                            ssem.at[buf_row]).wait()

                    @pl.when(blk > 0)
                    def _():
                        drain_scatter(0)

                    build_idx(0, 0)
                    issue(0)

                    def p2_body(cc, t2):
                        par = lax.rem(cc, 2)
                        nxt = 1 - par
                        g = lax.div(cc, CG)
                        rr = lax.rem(cc, CG)

                        @pl.when(cc < CPB - 1)
                        def _():
                            @pl.when((blk > 0) | (cc > 0))
                            def _():
                                drain_scatter(nxt)

                            build_idx(cc + 1, nxt)
                            issue(nxt)

                        if half == 1:
                            @pl.when(rr == 0)
                            def _():
                                pltpu.sync_copy(
                                    p_out.at[fc, s, blk, g], pexp)
                        for j in range(KC // 16):
                            dv = dst25[cc, pl.ds(16 * j, 16)]
                            dl = dv - base
                            ok = (dl >= 0) & (dl < HN)
                            sdx[par, pl.ds(16 * j, 16)] = jnp.where(
                                ok, dl, TRASH)
                            if half == 0:
                                sv = src25[cc, pl.ds(16 * j, 16)]
                                pexp[rr, pl.ds(16 * j, 16)] = leaky_exp(
                                    sv, dv, shc)
                        drain(par)

                        def sk(jj, tt):
                            pv = pexp[rr, pl.ds(16 * jj, 16)]
                            for k in range(16):
                                ps = pv[k]
                                kk = 16 * jj + k
                                for j in range(8):
                                    rows3[par * KC + kk,
                                          pl.ds(16 * j, 16)] = (
                                        rows3[par * KC + kk,
                                              pl.ds(16 * j, 16)] * ps)
                            return tt

                        lax.fori_loop(0, KC // 16, sk, 0)
                        pltpu.async_copy(rows3.at[pl.ds(par * KC, KC)],
                                         slab.at[sdx.at[par]],
                                         ssem.at[par], add=True)
                        if half == 0:
                            @pl.when(rr == CG - 1)
                            def _():
                                pltpu.sync_copy(
                                    pexp, p_out.at[fc, s, blk, g])
                        return t2

                    lax.fori_loop(0, CPB, p2_body, 0)
                    return t0

                lax.fori_loop(0, NBLK, blk_loop, 0)
                for b in range(2):
                    pltpu.make_async_copy(
                        rows3.at[pl.ds(b * KC, KC)],
                        slab.at[sdx.at[b]], ssem.at[b]).wait()

                plsc.subcore_barrier()
                pltpu.sync_copy(
                    slab.at[pl.ds(HRW * s, HRW)],
                    out_raw.at[fc, pl.ds(base + HRW * s, HRW)])
                plsc.subcore_barrier()

        for half in range(NP // HN):
            base = half * HN
            zero_own_slab()
            plsc.subcore_barrier()

            def dblk_loop(blk, t0, base=base):
                pltpu.sync_copy(dst4.at[s, blk], dst25)

                def dg_loop(g, t1):
                    for hl in range(NHPS):
                        pltpu.sync_copy(p_out.at[c * NHPS + hl, s, blk, g],
                                        pd[hl])

                    def d_body(rr, t2):
                        cc = CG * g + rr
                        for jj in range(KC // 16):
                            pvs = [pd[hl][rr, pl.ds(16 * jj, 16)]
                                   for hl in range(NHPS)]
                            for k in range(16):
                                kk = 16 * jj + k
                                for hl in range(NHPS):
                                    rows3[kk, pl.ds(16 * hl, 16)] = (
                                        jnp.full((16,), pvs[hl][k], f32))
                            dv = dst25[cc, pl.ds(16 * jj, 16)]
                            dl = dv - base
                            ok = (dl >= 0) & (dl < HN)
                            sdx[0, pl.ds(16 * jj, 16)] = jnp.where(
                                ok, dl, TRASH)
                        pltpu.sync_copy(rows3.at[pl.ds(0, KC)],
                                        slab.at[sdx.at[0]], add=True)
                        return t2

                    lax.fori_loop(0, CG, d_body, 0)
                    return t1

                lax.fori_loop(0, G, dg_loop, 0)
                return t0

            lax.fori_loop(0, NBLK, dblk_loop, 0)

            plsc.subcore_barrier()
            pltpu.sync_copy(
                slab.at[pl.ds(HRW * s, HRW)],
                den_out.at[c, pl.ds(base + HRW * s, HRW)])
            plsc.subcore_barrier()

    return k


@functools.lru_cache(maxsize=None)
def _sc_kernel():
    return _make_sc()


def _attn_pseudo(cols_src, cols_dst, dout):
    A = jnp.zeros((dout, 16), f32)
    for f in range(8):
        if cols_src[f] is not None:
            r0, v = cols_src[f]
            A = A.at[r0:r0 + v.shape[0], f].set(v)
        if cols_dst[f] is not None:
            r0, v = cols_dst[f]
            A = A.at[r0:r0 + v.shape[0], 8 + f].set(v)
    return A


def _attn_mat4(a_src, a_dst):
    cs = [(256 * (f // 2), a_src[f // 2]) for f in range(8)]
    cd = [(256 * (f // 2), a_dst[f // 2]) for f in range(8)]
    return _attn_pseudo(cs, cd, 1024)


def _attn_mat6(a_src, a_dst):
    cs = [(128 * f, a_src[f]) if f < 6 else None for f in range(8)]
    cd = [(128 * f, a_dst[f]) if f < 6 else None for f in range(8)]
    return _attn_pseudo(cs, cd, 1024)


def kernel(input_matrix, adj, W1, a1_src, a1_dst, W2, a2_src, a2_dst,
           W3, a3_src, a3_dst):
    src4 = adj[0].astype(jnp.int32).reshape(NS, NBLK, CPB, KC)
    dst4 = adj[1].astype(jnp.int32).reshape(NS, NBLK, CPB, KC)

    A1 = _attn_mat4(a1_src, a1_dst)
    A2 = _attn_mat4(a2_src, a2_dst)
    A3 = _attn_mat6(a3_src, a3_dst)
    W3p = jnp.pad(W3.reshape(1024, 6, 121),
                  ((0, 0), (0, 0), (0, 7)))
    W3p = jnp.pad(W3p.reshape(1024, 768), ((0, 0), (0, 256)))
    xp = jnp.pad(input_matrix, ((0, NP - N), (0, 0)))

    sc = _sc_kernel()

    def tview(a):
        return a.T.reshape(16 * NP)

    def nview(raw):
        return jnp.transpose(raw, (1, 0, 2)).reshape(NP, 1024)

    def dview(dcn):
        return jnp.concatenate([dcn[0], dcn[1]], axis=1)

    _, h1, asad1, sh1 = _tc_layer(xp, None, None, W1, A1)
    raw1, den1, _p = sc(h1.reshape(NP * 8, 128), tview(asad1),
                        sh1.reshape(16), src4, dst4)
    x1, h2, asad2, sh2 = _tc_layer(nview(raw1), dview(den1), None, W2, A2)
    raw2, den2, _p = sc(h2.reshape(NP * 8, 128), tview(asad2),
                        sh2.reshape(16), src4, dst4)
    _, h3, asad3, sh3 = _tc_layer(nview(raw2), dview(den2), x1, W3p, A3)
    raw3, den3, _p = sc(h3.reshape(NP * 8, 128), tview(asad3),
                        sh3.reshape(16), src4, dst4)
    out = _tc_final(nview(raw3), dview(den3))
    return out[:N, :121]


# speedup vs baseline: 11.5494x; 1.0687x over previous
import functools

import jax
import jax.numpy as jnp
from jax import lax
from jax.experimental import pallas as pl
from jax.experimental.pallas import tpu as pltpu
from jax.experimental.pallas import tpu_sc as plsc

N = 10000
NP = 10240
E = 160000
NS = 16
NC = 2
EPW = E // NS
KC = 80
NCH = EPW // KC
NBLK = 5
CPB = NCH // NBLK
NH = 8
NHPS = NH // NC
RPW = NP // NS
_ZCNT = (80,) * 8

f32 = jnp.float32


def _tc_layer(x_or_raw, den, res, W, A, bm=512):
    n, din = x_or_raw.shape
    dout = W.shape[1]
    prologue = den is not None

    def body(*refs):
        if prologue:
            if res is not None:
                raw_ref, den_ref, res_ref, w_ref, a_ref = refs[:5]
                xo_ref, h_ref, as_ref, sh_ref = refs[5:]
            else:
                raw_ref, den_ref, w_ref, a_ref = refs[:4]
                xo_ref, h_ref, as_ref, sh_ref = refs[4:]
            d = den_ref[...]
            parts = []
            for hh in range(8):
                col = 128 * (hh // 4) + 16 * (hh % 4)
                parts.append(raw_ref[:, hh * 128:(hh + 1) * 128]
                             / (d[:, col:col + 1] + 1e-16))
            v = jnp.concatenate(parts, axis=1)
            x = jnp.where(v > 0.0, v, jnp.exp(v) - 1.0)
            if res is not None:
                x = x + res_ref[...]
            xo_ref[...] = x
        else:
            x_ref, w_ref, a_ref, h_ref, as_ref, sh_ref = refs
            x = x_ref[...]
        i = pl.program_id(0)
        h = jnp.dot(x, w_ref[...], preferred_element_type=f32)
        h_ref[...] = h
        asad = jnp.dot(h, a_ref[...], preferred_element_type=f32)
        as_ref[...] = asad
        m = jnp.max(asad, axis=0, keepdims=True)

        @pl.when(i == 0)
        def _():
            sh_ref[...] = m

        @pl.when(i != 0)
        def _():
            sh_ref[...] = jnp.maximum(sh_ref[...], m)

    grid = (n // bm,)
    in_specs = []
    ins = []
    if prologue:
        in_specs.append(pl.BlockSpec((bm, din), lambda i: (i, 0)))
        ins.append(x_or_raw)
        in_specs.append(pl.BlockSpec((bm, 256), lambda i: (i, 0)))
        ins.append(den)
        if res is not None:
            in_specs.append(pl.BlockSpec((bm, din), lambda i: (i, 0)))
            ins.append(res)
    else:
        in_specs.append(pl.BlockSpec((bm, din), lambda i: (i, 0)))
        ins.append(x_or_raw)
    in_specs.append(pl.BlockSpec((din, dout), lambda i: (0, 0)))
    ins.append(W)
    in_specs.append(pl.BlockSpec((dout, 16), lambda i: (0, 0)))
    ins.append(A)

    out_specs = []
    out_shape = []
    if prologue:
        out_specs.append(pl.BlockSpec((bm, din), lambda i: (i, 0)))
        out_shape.append(jax.ShapeDtypeStruct((n, din), f32))
    out_specs.append(pl.BlockSpec((bm, dout), lambda i: (i, 0)))
    out_shape.append(jax.ShapeDtypeStruct((n, dout), f32))
    out_specs.append(pl.BlockSpec((bm, 16), lambda i: (i, 0)))
    out_shape.append(jax.ShapeDtypeStruct((n, 16), f32))
    out_specs.append(pl.BlockSpec((1, 16), lambda i: (0, 0)))
    out_shape.append(jax.ShapeDtypeStruct((1, 16), f32))

    outs = pl.pallas_call(body, grid=grid, in_specs=in_specs,
                          out_specs=out_specs, out_shape=out_shape)(*ins)
    if prologue:
        x, h, asad, shift = outs
    else:
        h, asad, shift = outs
        x = x_or_raw
    return x, h, asad, shift


def _tc_final(raw, den, bm=512):
    def body(raw_ref, den_ref, o_ref):
        d = den_ref[...]
        acc = raw_ref[:, 0:128] / (d[:, 0:1] + 1e-16)
        for hh in range(1, 6):
            col = 128 * (hh // 3) + 16 * (hh % 3)
            acc = acc + (raw_ref[:, hh * 128:(hh + 1) * 128]
                         / (d[:, col:col + 1] + 1e-16))
        o_ref[...] = acc * (1.0 / 6.0)

    return pl.pallas_call(
        body, grid=(NP // bm,),
        in_specs=[pl.BlockSpec((bm, 1024), lambda i: (i, 0)),
                  pl.BlockSpec((bm, 256), lambda i: (i, 0))],
        out_specs=pl.BlockSpec((bm, 128), lambda i: (i, 0)),
        out_shape=jax.ShapeDtypeStruct((NP, 128), f32),
    )(raw, den)


HN = 5120
TRASH = HN
HRW = HN // NS


def _make_sc():
    mesh = plsc.VectorSubcoreMesh(core_axis_name="c", subcore_axis_name="s")
    G = 5
    CG = CPB // G

    @functools.partial(
        pl.kernel,
        out_type=[jax.ShapeDtypeStruct((NH, NP, 128), f32),
                  jax.ShapeDtypeStruct((NC, NP, 128), f32),
                  jax.ShapeDtypeStruct((NH, NS, NBLK, G, CG, KC), f32)],
        mesh=mesh,
        compiler_params=pltpu.CompilerParams(needs_layout_passes=False),
        scratch_types=[
            pltpu.VMEM((CPB, KC), jnp.int32),
            pltpu.VMEM((CPB, KC), jnp.int32),
            pltpu.VMEM((2, KC), jnp.int32),
            pltpu.VMEM((CG, KC), f32),
            pltpu.VMEM((CG, KC), f32),
            pltpu.VMEM((CG, KC), f32),
            pltpu.VMEM((CG, KC), f32),
            pltpu.VMEM((CG, KC), f32),
            pltpu.VMEM((NP,), f32),
            pltpu.VMEM((NP,), f32),
            pltpu.VMEM((2, KC), jnp.int32),
            pltpu.VMEM((2 * KC, 128), f32),
            pltpu.VMEM((16,), f32),
            pltpu.VMEM((16,), jnp.int32),
            pltpu.SemaphoreType.DMA((2,)),
            pltpu.SemaphoreType.DMA((2,)),
            pltpu.VMEM_SHARED((HN + 8, 128), f32),
        ],
    )
    def k(h_flat, asadT, shift_in, src4, dst4, nact16, out_raw, den_out,
          p_out, src25, dst25, sdx, pexp, pd0, pd1, pd2, pd3,
          as_buf, ad_buf, idx3, rows3, shiftv, nbuf, gsem, ssem, slab):
        c = lax.axis_index("c")
        s = lax.axis_index("s")
        pltpu.sync_copy(shift_in, shiftv)
        pltpu.sync_copy(nact16, nbuf)
        nact = nbuf[pl.ds(0, 16)][0]
        pib = "promise_in_bounds"
        pd = [pd0, pd1, pd2, pd3]

        def zero_rows(r, t):
            for j in range(8):
                rows3[r, pl.ds(16 * j, 16)] = jnp.zeros((16,), f32)
            return t

        def zero_own_slab():
            lax.fori_loop(0, KC, zero_rows, 0)
            off = 0
            for _ in range(HRW // KC):
                pltpu.sync_copy(rows3.at[pl.ds(0, KC)],
                                slab.at[pl.ds(HRW * s + off, KC)])
                off += KC

            @pl.when(s == NS - 1)
            def _():
                pltpu.sync_copy(rows3.at[pl.ds(0, 8)],
                                slab.at[pl.ds(HN, 8)])

        def leaky_exp(sv, dv, shc):
            a1 = plsc.load_gather(as_buf, [sv])
            a2 = plsc.load_gather(ad_buf, [dv])
            e = a1 + a2
            e = jnp.where(e >= 0.0, e, 0.2 * e)
            return jnp.exp(e - shc)

        for fci in range(NHPS):
            fc = c * nact + fci
            act = fci < nact
            fcv = jnp.full((16,), fc, jnp.int32)
            shv = shiftv[pl.ds(0, 16)]
            shc = jnp.maximum(shv.at[fcv].get(mode=pib)
                              + shv.at[fcv + 8].get(mode=pib), 0.0)

            @pl.when(act)
            def _():
                pltpu.sync_copy(asadT.at[pl.ds(fc * NP, NP)], as_buf)
                pltpu.sync_copy(asadT.at[pl.ds((8 + fc) * NP, NP)], ad_buf)

            for half in range(NP // HN):
                base = half * HN

                @pl.when(act)
                def _():
                    zero_own_slab()

                plsc.subcore_barrier()

                def blk_loop(blk, t0, half=half, fc=fc, shc=shc, base=base):
                    pltpu.sync_copy(src4.at[s, blk], src25)
                    pltpu.sync_copy(dst4.at[s, blk], dst25)

                    def build_idx(cc, buf_row):
                        for j in range(KC // 16):
                            sv = src25[cc, pl.ds(16 * j, 16)]
                            idx3[buf_row, pl.ds(16 * j, 16)] = sv * NH + fc

                    def issue(buf_row):
                        pltpu.async_copy(
                            h_flat.at[idx3.at[buf_row]],
                            rows3.at[pl.ds(buf_row * KC, KC)],
                            gsem.at[buf_row])

                    def drain(buf_row):
                        pltpu.make_async_copy(
                            h_flat.at[idx3.at[buf_row]],
                            rows3.at[pl.ds(buf_row * KC, KC)],
                            gsem.at[buf_row]).wait()

                    def drain_scatter(buf_row):
                        pltpu.make_async_copy(
                            rows3.at[pl.ds(buf_row * KC, KC)],
                            slab.at[sdx.at[buf_row]],
                            ssem.at[buf_row]).wait()

                    @pl.when(blk > 0)
                    def _():
                        drain_scatter(0)

                    build_idx(0, 0)
                    issue(0)

                    def p2_body(cc, t2):
                        par = lax.rem(cc, 2)
                        nxt = 1 - par
                        g = lax.div(cc, CG)
                        rr = lax.rem(cc, CG)

                        @pl.when(cc < CPB - 1)
                        def _():
                            @pl.when((blk > 0) | (cc > 0))
                            def _():
                                drain_scatter(nxt)

                            build_idx(cc + 1, nxt)
                            issue(nxt)

                        if half == 1:
                            @pl.when(rr == 0)
                            def _():
                                pltpu.sync_copy(
                                    p_out.at[fc, s, blk, g], pexp)
                        for j in range(KC // 16):
                            dv = dst25[cc, pl.ds(16 * j, 16)]
                            dl = dv - base
                            ok = (dl >= 0) & (dl < HN)
                            sdx[par, pl.ds(16 * j, 16)] = jnp.where(
                                ok, dl, TRASH)
                            if half == 0:
                                sv = src25[cc, pl.ds(16 * j, 16)]
                                pexp[rr, pl.ds(16 * j, 16)] = leaky_exp(
                                    sv, dv, shc)
                        drain(par)

                        def sk(jj, tt):
                            pv = pexp[rr, pl.ds(16 * jj, 16)]
                            for k in range(16):
                                ps = pv[k]
                                kk = 16 * jj + k
                                for j in range(8):
                                    rows3[par * KC + kk,
                                          pl.ds(16 * j, 16)] = (
                                        rows3[par * KC + kk,
                                              pl.ds(16 * j, 16)] * ps)
                            return tt

                        lax.fori_loop(0, KC // 16, sk, 0)
                        pltpu.async_copy(rows3.at[pl.ds(par * KC, KC)],
                                         slab.at[sdx.at[par]],
                                         ssem.at[par], add=True)
                        if half == 0:
                            @pl.when(rr == CG - 1)
                            def _():
                                pltpu.sync_copy(
                                    pexp, p_out.at[fc, s, blk, g])
                        return t2

                    lax.fori_loop(0, CPB, p2_body, 0)
                    return t0

                @pl.when(act)
                def _():
                    lax.fori_loop(0, NBLK, blk_loop, 0)
                    for b in range(2):
                        pltpu.make_async_copy(
                            rows3.at[pl.ds(b * KC, KC)],
                            slab.at[sdx.at[b]], ssem.at[b]).wait()

                plsc.subcore_barrier()

                @pl.when(act)
                def _():
                    pltpu.sync_copy(
                        slab.at[pl.ds(HRW * s, HRW)],
                        out_raw.at[fc, pl.ds(base + HRW * s, HRW)])

                plsc.subcore_barrier()

        for half in range(NP // HN):
            base = half * HN
            zero_own_slab()
            plsc.subcore_barrier()

            def dblk_loop(blk, t0, base=base):
                pltpu.sync_copy(dst4.at[s, blk], dst25)

                def dg_loop(g, t1):
                    for hl in range(NHPS):
                        hs = c * nact + jnp.minimum(hl, nact - 1)
                        pltpu.sync_copy(p_out.at[hs, s, blk, g], pd[hl])

                    def d_body(rr, t2):
                        cc = CG * g + rr
                        for jj in range(KC // 16):
                            pvs = [pd[hl][rr, pl.ds(16 * jj, 16)]
                                   for hl in range(NHPS)]
                            for k in range(16):
                                kk = 16 * jj + k
                                for hl in range(NHPS):
                                    rows3[kk, pl.ds(16 * hl, 16)] = (
                                        jnp.full((16,), pvs[hl][k], f32))
                            dv = dst25[cc, pl.ds(16 * jj, 16)]
                            dl = dv - base
                            ok = (dl >= 0) & (dl < HN)
                            sdx[0, pl.ds(16 * jj, 16)] = jnp.where(
                                ok, dl, TRASH)
                        pltpu.sync_copy(rows3.at[pl.ds(0, KC)],
                                        slab.at[sdx.at[0]], add=True)
                        return t2

                    lax.fori_loop(0, CG, d_body, 0)
                    return t1

                lax.fori_loop(0, G, dg_loop, 0)
                return t0

            lax.fori_loop(0, NBLK, dblk_loop, 0)

            plsc.subcore_barrier()
            pltpu.sync_copy(
                slab.at[pl.ds(HRW * s, HRW)],
                den_out.at[c, pl.ds(base + HRW * s, HRW)])
            plsc.subcore_barrier()

    return k


@functools.lru_cache(maxsize=None)
def _sc_kernel():
    return _make_sc()


def _attn_pseudo(cols_src, cols_dst, dout):
    A = jnp.zeros((dout, 16), f32)
    for f in range(8):
        if cols_src[f] is not None:
            r0, v = cols_src[f]
            A = A.at[r0:r0 + v.shape[0], f].set(v)
        if cols_dst[f] is not None:
            r0, v = cols_dst[f]
            A = A.at[r0:r0 + v.shape[0], 8 + f].set(v)
    return A


def _attn_mat4(a_src, a_dst):
    cs = [(256 * (f // 2), a_src[f // 2]) for f in range(8)]
    cd = [(256 * (f // 2), a_dst[f // 2]) for f in range(8)]
    return _attn_pseudo(cs, cd, 1024)


def _attn_mat6(a_src, a_dst):
    cs = [(128 * f, a_src[f]) if f < 6 else None for f in range(8)]
    cd = [(128 * f, a_dst[f]) if f < 6 else None for f in range(8)]
    return _attn_pseudo(cs, cd, 1024)


def kernel(input_matrix, adj, W1, a1_src, a1_dst, W2, a2_src, a2_dst,
           W3, a3_src, a3_dst):
    src4 = adj[0].astype(jnp.int32).reshape(NS, NBLK, CPB, KC)
    dst4 = adj[1].astype(jnp.int32).reshape(NS, NBLK, CPB, KC)

    A1 = _attn_mat4(a1_src, a1_dst)
    A2 = _attn_mat4(a2_src, a2_dst)
    A3 = _attn_mat6(a3_src, a3_dst)
    W3p = jnp.pad(W3.reshape(1024, 6, 121),
                  ((0, 0), (0, 0), (0, 7)))
    W3p = jnp.pad(W3p.reshape(1024, 768), ((0, 0), (0, 256)))
    xp = jnp.pad(input_matrix, ((0, NP - N), (0, 0)))

    sc = _sc_kernel()

    def tview(a):
        return a.T.reshape(16 * NP)

    def nview(raw):
        return jnp.transpose(raw, (1, 0, 2)).reshape(NP, 1024)

    def dview(dcn):
        return jnp.concatenate([dcn[0], dcn[1]], axis=1)

    n4 = jnp.full((16,), 4, jnp.int32)
    n3 = jnp.full((16,), 3, jnp.int32)

    _, h1, asad1, sh1 = _tc_layer(xp, None, None, W1, A1)
    raw1, den1, _p = sc(h1.reshape(NP * 8, 128), tview(asad1),
                        sh1.reshape(16), src4, dst4, n4)
    x1, h2, asad2, sh2 = _tc_layer(nview(raw1), dview(den1), None, W2, A2)
    raw2, den2, _p = sc(h2.reshape(NP * 8, 128), tview(asad2),
                        sh2.reshape(16), src4, dst4, n4)
    _, h3, asad3, sh3 = _tc_layer(nview(raw2), dview(den2), x1, W3p, A3)
    raw3, den3, _p = sc(h3.reshape(NP * 8, 128), tview(asad3),
                        sh3.reshape(16), src4, dst4, n3)
    out = _tc_final(nview(raw3), dview(den3))
    return out[:N, :121]
